# baseline (speedup 1.0000x reference)
import jax
import jax.numpy as jnp
from jax.experimental import pallas as pl
from jax.experimental.pallas import tpu as pltpu

_TN = 2048
_NBUF = 6
_NIN = 3
_N = 100000
_NFULL = _N // _TN


def _dot(x, m):
    return jax.lax.dot_general(
        x,
        m,
        dimension_numbers=(((1,), (1,)), ((), ())),
        preferred_element_type=jnp.float32,
    )


def _main_kernel(x_ref, m_hbm, o_hbm, mbuf, obuf, insem, outsem):
    def in_copy(t):
        return pltpu.make_async_copy(
            m_hbm.at[pl.ds(t * _TN, _TN), :],
            mbuf.at[t % _NIN],
            insem.at[t % _NIN],
        )

    def out_copy(t):
        return pltpu.make_async_copy(
            obuf.at[t % _NBUF],
            o_hbm.at[:, pl.ds(t * _TN, _TN)],
            outsem.at[t % _NBUF],
        )

    for t in range(_NIN):
        in_copy(t).start(priority=t % 2)
    for t in range(_NFULL):
        in_copy(t).wait()
        if t >= _NBUF:
            out_copy(t - _NBUF).wait()
        obuf[t % _NBUF] = _dot(x_ref[...], mbuf[t % _NIN])
        if t + _NIN < _NFULL:
            in_copy(t + _NIN).start(priority=t % 2)
        out_copy(t).start(priority=t % 2)
    for t in range(_NFULL - _NBUF, _NFULL):
        out_copy(t).wait()


def _tail_kernel(x_ref, m_ref, o_aliased, o_ref):
    del o_aliased
    o_ref[...] = _dot(x_ref[...], m_ref[...])


def kernel(inputs, targets, mem, epoch):
    del targets, epoch
    m, k = inputs.shape
    n = mem.shape[0]
    main = pl.pallas_call(
        _main_kernel,
        in_specs=[
            pl.BlockSpec((m, k), lambda: (0, 0)),
            pl.BlockSpec(memory_space=pltpu.MemorySpace.HBM),
        ],
        out_specs=pl.BlockSpec(memory_space=pltpu.MemorySpace.HBM),
        out_shape=jax.ShapeDtypeStruct((m, n), jnp.float32),
        scratch_shapes=[
            pltpu.VMEM((_NIN, _TN, k), jnp.float32),
            pltpu.VMEM((_NBUF, m, _TN), jnp.float32),
            pltpu.SemaphoreType.DMA((_NIN,)),
            pltpu.SemaphoreType.DMA((_NBUF,)),
        ],
    )(inputs, mem)
    return pl.pallas_call(
        _tail_kernel,
        grid=(1,),
        in_specs=[
            pl.BlockSpec((m, k), lambda i: (0, 0)),
            pl.BlockSpec((_TN, k), lambda i: (_NFULL, 0)),
            pl.BlockSpec(memory_space=pltpu.MemorySpace.HBM),
        ],
        out_specs=pl.BlockSpec((m, _TN), lambda i: (0, _NFULL)),
        out_shape=jax.ShapeDtypeStruct((m, n), jnp.float32),
        input_output_aliases={2: 0},
    )(inputs, mem, main)
